# Initial kernel scaffold; baseline (speedup 1.0000x reference)
import jax
import jax.numpy as jnp
from jax.experimental import pallas as pl


def kernel(inputs, params, adj):
    raise NotImplementedError("write your pallas kernel here")


# speedup vs baseline: 8.7285x; 8.7285x over previous
import jax
import jax.numpy as jnp
from jax.experimental import pallas as pl

_N = 256
_B = 32
_P2 = _B // 2
_SEQ = 12
_HOR = 12
_U = 64
_M = 3
_F32 = jnp.float32


def _to_wide(h):
    return jnp.concatenate(
        [h[p * _N:(p + 1) * _N, :] for p in range(_P2)], axis=1)


def _to_pm(w):
    return jnp.concatenate(
        [w[:, p * 2 * _U:(p + 1) * 2 * _U] for p in range(_P2)], axis=0)


def _cheb_wide(scat, xw):
    y = jnp.dot(scat, xw, preferred_element_type=_F32)
    return y[:_N], y[_N:]


def _parts_cat(p0, w1, w2):
    return jnp.concatenate([p0, _to_pm(w1), _to_pm(w2)], axis=1)


def _cell(scat, xcat, hpm, hw, wg, bg, wc, bc):
    y1, y2 = _cheb_wide(scat, hw)
    hcat = _parts_cat(hpm, y1, y2)
    val = jnp.dot(jnp.concatenate([xcat, hcat], axis=1), wg,
                  preferred_element_type=_F32) + bg
    val = jax.nn.sigmoid(val)
    r = jnp.concatenate([val[:, 0:_U], val[:, 2 * _U:3 * _U]], axis=1)
    u = jnp.concatenate([val[:, _U:2 * _U], val[:, 3 * _U:]], axis=1)
    rh = r * hpm
    z1, z2 = _cheb_wide(scat, _to_wide(rh))
    rcat = _parts_cat(rh, z1, z2)
    c = jnp.tanh(jnp.dot(jnp.concatenate([xcat, rcat], axis=1), wc,
                         preferred_element_type=_F32) + bc)
    hn = u * hpm + (1.0 - u) * c
    return hn, _to_wide(hn)


def _dcrnn_body(scat_ref, x_ref, g_ref, pwc_ref,
                e0wxg, e0whg, e0bg, e0wxc, e0whc, e0bc,
                e1wxg, e1whg, e1bg, e1wxc, e1whc, e1bc,
                d0wxg, d0whg, d0bg, d0wxc, d0whc, d0bc,
                d1wxg, d1whg, d1bg, d1wxc, d1whc, d1bc,
                pb_ref, out_ref):
    scat = scat_ref[...]

    g = g_ref[...]
    pwc = pwc_ref[...]
    e0 = (jnp.concatenate([e0wxg[...], e0whg[...]], axis=0), e0bg[...],
          jnp.concatenate([e0wxc[...], e0whc[...]], axis=0), e0bc[...])
    e1 = (jnp.concatenate([e1wxg[...], e1whg[...]], axis=0), e1bg[...],
          jnp.concatenate([e1wxc[...], e1whc[...]], axis=0), e1bc[...])
    d0 = (jnp.concatenate([d0wxg[...], d0whg[...]], axis=0), d0bg[...],
          jnp.concatenate([d0wxc[...], d0whc[...]], axis=0), d0bc[...])
    d1 = (jnp.concatenate([d1wxg[...], d1whg[...]], axis=0), d1bg[...],
          jnp.concatenate([d1wxc[...], d1whc[...]], axis=0), d1bc[...])
    pb = pb_ref[...]

    def enc_step(t, carry):
        h0p, h0w, h1p, h1w = carry
        xc = x_ref[pl.ds(t * _N, _N), :]
        xw = jnp.dot(xc, g, preferred_element_type=_F32)
        y1, y2 = _cheb_wide(scat, xw)
        xcat = _parts_cat(_to_pm(xw), y1, y2)
        h0p, h0w = _cell(scat, xcat, h0p, h0w, *e0)
        y1, y2 = _cheb_wide(scat, h0w)
        xcat1 = _parts_cat(h0p, y1, y2)
        h1p, h1w = _cell(scat, xcat1, h1p, h1w, *e1)
        return h0p, h0w, h1p, h1w

    hp0 = jnp.zeros((_P2 * _N, 2 * _U), _F32)
    hw0 = jnp.zeros((_N, _B * _U), _F32)
    h0p, h0w, h1p, h1w = jax.lax.fori_loop(
        0, _SEQ, enc_step, (hp0, hw0, hp0, hw0))

    def dec_step(t, carry):
        h0p, h0w, h1p, h1w, dip, diw = carry
        y1, y2 = _cheb_wide(scat, diw)
        xcat = _parts_cat(dip, y1, y2)
        h0p, h0w = _cell(scat, xcat, h0p, h0w, *d0)
        y1, y2 = _cheb_wide(scat, h0w)
        xcat1 = _parts_cat(h0p, y1, y2)
        h1p, h1w = _cell(scat, xcat1, h1p, h1w, *d1)
        oc = jnp.dot(h1w, pwc, preferred_element_type=_F32) + pb
        out_ref[pl.ds(t * _N, _N), :] = oc
        diw = jnp.dot(oc, g, preferred_element_type=_F32)
        return h0p, h0w, h1p, h1w, _to_pm(diw), diw

    jax.lax.fori_loop(0, _HOR, dec_step, (h0p, h0w, h1p, h1w, hp0, hw0))


def _blkdiag2(w):
    z = jnp.zeros_like(w)
    return jnp.concatenate([jnp.concatenate([w, z], axis=1),
                            jnp.concatenate([z, w], axis=1)], axis=0)


def _prep(p, d):
    wg = p["Wg"].reshape(d + _U, _M, 2 * _U)
    wc = p["Wc"].reshape(d + _U, _M, _U)
    wxg = jnp.transpose(wg[:d], (1, 0, 2))
    wxc = jnp.transpose(wc[:d], (1, 0, 2))
    if d == 1:
        pad = ((0, 0), (0, _U - 1), (0, 0))
        wxg = jnp.pad(wxg, pad)
        wxc = jnp.pad(wxc, pad)
    whg = jnp.transpose(wg[d:], (1, 0, 2))
    whc = jnp.transpose(wc[d:], (1, 0, 2))
    wxg_cat = jnp.concatenate([_blkdiag2(wxg[m]) for m in range(_M)], axis=0)
    whg_cat = jnp.concatenate([_blkdiag2(whg[m]) for m in range(_M)], axis=0)
    wxc_cat = jnp.concatenate([_blkdiag2(wxc[m]) for m in range(_M)], axis=0)
    whc_cat = jnp.concatenate([_blkdiag2(whc[m]) for m in range(_M)], axis=0)
    bg2 = jnp.tile(p["bg"], 2).reshape(1, 4 * _U)
    bc2 = jnp.tile(p["bc"], 2).reshape(1, 2 * _U)
    return (wxg_cat, whg_cat, bg2, wxc_cat, whc_cat, bc2)


def kernel(inputs, params, adj):
    x_c = jnp.transpose(inputs, (0, 2, 1)).reshape(_SEQ * _N, _B)
    g = (jnp.arange(_B)[:, None] * _U == jnp.arange(_B * _U)[None, :])
    g = g.astype(_F32)
    dsum = jnp.sum(adj, axis=1)
    dis = jnp.where(dsum > 0, 1.0 / jnp.sqrt(dsum), 0.0)
    s1 = -(dis[:, None] * adj * dis[None, :])
    s2 = 2.0 * (s1 @ s1) - jnp.eye(_N, dtype=_F32)
    scat = jnp.concatenate([s1, s2], axis=0)
    pwc = (jnp.eye(_B, dtype=_F32)[:, None, :]
           * params["proj"]["W"][None, :, 0:1]).reshape(_B * _U, _B)
    args = [scat, x_c, g, pwc]
    for p, d in ((params["enc"][0], 1), (params["enc"][1], _U),
                 (params["dec"][0], 1), (params["dec"][1], _U)):
        args.extend(_prep(p, d))
    args.append(params["proj"]["b"].reshape(1, 1))
    out = pl.pallas_call(
        _dcrnn_body,
        out_shape=jax.ShapeDtypeStruct((_HOR * _N, _B), _F32),
    )(*args)
    out = out.reshape(_HOR, _N, _B)
    return jnp.transpose(out, (0, 2, 1))


# speedup vs baseline: 10.3789x; 1.1891x over previous
import jax
import jax.numpy as jnp
from jax.experimental import pallas as pl

_N = 256
_B = 32
_P2 = _B // 2
_SEQ = 12
_HOR = 12
_U = 64
_M = 3
_F32 = jnp.float32


def _to_wide(h):
    return jnp.concatenate(
        [h[p * _N:(p + 1) * _N, :] for p in range(_P2)], axis=1)


def _to_pm(w, piece=2 * _U):
    return jnp.concatenate(
        [w[:, p * piece:(p + 1) * piece] for p in range(_P2)], axis=0)


def _cheb_wide(scat, xw):
    y = jnp.dot(scat, xw, preferred_element_type=_F32)
    return y[:_N], y[_N:]


def _parts_cat(p0, w1, w2):
    return jnp.concatenate([p0, _to_pm(w1), _to_pm(w2)], axis=1)


def _cell0(scat, xk, hpm, hw, wgh, bg, hgk, wch, bc, hck):
    y1, y2 = _cheb_wide(scat, hw)
    hcat = _parts_cat(hpm, y1, y2)
    cg = jnp.dot(xk, hgk, preferred_element_type=_F32)
    val = (jnp.dot(hcat, wgh, preferred_element_type=_F32)
           + _to_pm(cg, 4 * _U) + bg)
    val = jax.nn.sigmoid(val)
    r = jnp.concatenate([val[:, 0:_U], val[:, 2 * _U:3 * _U]], axis=1)
    u = jnp.concatenate([val[:, _U:2 * _U], val[:, 3 * _U:]], axis=1)
    rh = r * hpm
    z1, z2 = _cheb_wide(scat, _to_wide(rh))
    rcat = _parts_cat(rh, z1, z2)
    cc = jnp.dot(xk, hck, preferred_element_type=_F32)
    c = jnp.tanh(jnp.dot(rcat, wch, preferred_element_type=_F32)
                 + _to_pm(cc) + bc)
    hn = u * hpm + (1.0 - u) * c
    return hn, _to_wide(hn)


def _cell1(scat, xcat, hpm, hw, wg, bg, wc, bc):
    y1, y2 = _cheb_wide(scat, hw)
    hcat = _parts_cat(hpm, y1, y2)
    val = jnp.dot(jnp.concatenate([xcat, hcat], axis=1), wg,
                  preferred_element_type=_F32) + bg
    val = jax.nn.sigmoid(val)
    r = jnp.concatenate([val[:, 0:_U], val[:, 2 * _U:3 * _U]], axis=1)
    u = jnp.concatenate([val[:, _U:2 * _U], val[:, 3 * _U:]], axis=1)
    rh = r * hpm
    z1, z2 = _cheb_wide(scat, _to_wide(rh))
    rcat = _parts_cat(rh, z1, z2)
    c = jnp.tanh(jnp.dot(jnp.concatenate([xcat, rcat], axis=1), wc,
                         preferred_element_type=_F32) + bc)
    hn = u * hpm + (1.0 - u) * c
    return hn, _to_wide(hn)


def _xk(scat, xc):
    y = jnp.dot(scat, xc, preferred_element_type=_F32)
    return jnp.concatenate([xc, y[:_N], y[_N:]], axis=1)


def _dcrnn_body(scat_ref, x_ref, pwc_ref,
                e0wgh, e0bg, e0hgk, e0wch, e0bc, e0hck,
                e1wg, e1bg, e1wc, e1bc,
                d0wgh, d0bg, d0hgk, d0wch, d0bc, d0hck,
                d1wg, d1bg, d1wc, d1bc,
                pb_ref, out_ref):
    scat = scat_ref[...]
    pwc = pwc_ref[...]
    e0 = (e0wgh[...], e0bg[...], e0hgk[...], e0wch[...], e0bc[...], e0hck[...])
    e1 = (e1wg[...], e1bg[...], e1wc[...], e1bc[...])
    d0 = (d0wgh[...], d0bg[...], d0hgk[...], d0wch[...], d0bc[...], d0hck[...])
    d1 = (d1wg[...], d1bg[...], d1wc[...], d1bc[...])
    pb = pb_ref[...]

    def enc_step(t, carry):
        h0p, h0w, h1p, h1w = carry
        xc = x_ref[pl.ds(t * _N, _N), :]
        xk = _xk(scat, xc)
        h0p, h0w = _cell0(scat, xk, h0p, h0w, *e0)
        y1, y2 = _cheb_wide(scat, h0w)
        xcat1 = _parts_cat(h0p, y1, y2)
        h1p, h1w = _cell1(scat, xcat1, h1p, h1w, *e1)
        return h0p, h0w, h1p, h1w

    hp0 = jnp.zeros((_P2 * _N, 2 * _U), _F32)
    hw0 = jnp.zeros((_N, _B * _U), _F32)
    h0p, h0w, h1p, h1w = jax.lax.fori_loop(
        0, _SEQ, enc_step, (hp0, hw0, hp0, hw0))

    def dec_step(t, carry):
        h0p, h0w, h1p, h1w, oc = carry
        xk = _xk(scat, oc)
        h0p, h0w = _cell0(scat, xk, h0p, h0w, *d0)
        y1, y2 = _cheb_wide(scat, h0w)
        xcat1 = _parts_cat(h0p, y1, y2)
        h1p, h1w = _cell1(scat, xcat1, h1p, h1w, *d1)
        oc = jnp.dot(h1w, pwc, preferred_element_type=_F32) + pb
        out_ref[pl.ds(t * _N, _N), :] = oc
        return h0p, h0w, h1p, h1w, oc

    oc0 = jnp.zeros((_N, _B), _F32)
    jax.lax.fori_loop(0, _HOR, dec_step, (h0p, h0w, h1p, h1w, oc0))


def _blkdiag2(w):
    z = jnp.zeros_like(w)
    return jnp.concatenate([jnp.concatenate([w, z], axis=1),
                            jnp.concatenate([z, w], axis=1)], axis=0)


def _prep0(p):
    wg = p["Wg"].reshape(1 + _U, _M, 2 * _U)
    wc = p["Wc"].reshape(1 + _U, _M, _U)
    whg = jnp.transpose(wg[1:], (1, 0, 2))
    whc = jnp.transpose(wc[1:], (1, 0, 2))
    wgh_cat = jnp.concatenate([_blkdiag2(whg[m]) for m in range(_M)], axis=0)
    wch_cat = jnp.concatenate([_blkdiag2(whc[m]) for m in range(_M)], axis=0)
    eye = jnp.eye(_B, dtype=_F32)
    hgk = jnp.concatenate(
        [(eye[:, :, None] * wg[0, m][None, None, :]).reshape(_B, _B * 2 * _U)
         for m in range(_M)], axis=0)
    hck = jnp.concatenate(
        [(eye[:, :, None] * wc[0, m][None, None, :]).reshape(_B, _B * _U)
         for m in range(_M)], axis=0)
    bg2 = jnp.tile(p["bg"], 2).reshape(1, 4 * _U)
    bc2 = jnp.tile(p["bc"], 2).reshape(1, 2 * _U)
    return (wgh_cat, bg2, hgk, wch_cat, bc2, hck)


def _prep1(p):
    wg = p["Wg"].reshape(2 * _U, _M, 2 * _U)
    wc = p["Wc"].reshape(2 * _U, _M, _U)
    wxg = jnp.transpose(wg[:_U], (1, 0, 2))
    wxc = jnp.transpose(wc[:_U], (1, 0, 2))
    whg = jnp.transpose(wg[_U:], (1, 0, 2))
    whc = jnp.transpose(wc[_U:], (1, 0, 2))
    wg_cat = jnp.concatenate(
        [_blkdiag2(wxg[m]) for m in range(_M)]
        + [_blkdiag2(whg[m]) for m in range(_M)], axis=0)
    wc_cat = jnp.concatenate(
        [_blkdiag2(wxc[m]) for m in range(_M)]
        + [_blkdiag2(whc[m]) for m in range(_M)], axis=0)
    bg2 = jnp.tile(p["bg"], 2).reshape(1, 4 * _U)
    bc2 = jnp.tile(p["bc"], 2).reshape(1, 2 * _U)
    return (wg_cat, bg2, wc_cat, bc2)


def kernel(inputs, params, adj):
    x_c = jnp.transpose(inputs, (0, 2, 1)).reshape(_SEQ * _N, _B)
    dsum = jnp.sum(adj, axis=1)
    dis = jnp.where(dsum > 0, 1.0 / jnp.sqrt(dsum), 0.0)
    s1 = -(dis[:, None] * adj * dis[None, :])
    s2 = 2.0 * (s1 @ s1) - jnp.eye(_N, dtype=_F32)
    scat = jnp.concatenate([s1, s2], axis=0)
    pwc = (jnp.eye(_B, dtype=_F32)[:, None, :]
           * params["proj"]["W"][None, :, 0:1]).reshape(_B * _U, _B)
    args = [scat, x_c, pwc]
    args.extend(_prep0(params["enc"][0]))
    args.extend(_prep1(params["enc"][1]))
    args.extend(_prep0(params["dec"][0]))
    args.extend(_prep1(params["dec"][1]))
    args.append(params["proj"]["b"].reshape(1, 1))
    out = pl.pallas_call(
        _dcrnn_body,
        out_shape=jax.ShapeDtypeStruct((_HOR * _N, _B), _F32),
    )(*args)
    out = out.reshape(_HOR, _N, _B)
    return jnp.transpose(out, (0, 2, 1))
